# baseline (speedup 1.0000x reference)
import functools

import jax
import jax.numpy as jnp
from jax import lax
from jax.experimental import pallas as pl
from jax.experimental.pallas import tpu as pltpu
from jax.experimental.pallas import tpu_sc as plsc

NB = 1000
DUMP = NB * NB
TBL = NB * NB + 16
N_GCHUNK = 246
TILE_LIN = N_GCHUNK * 128
LIN_PAD = 32 * TILE_LIN
N_SUB = 3
SUB_C = N_GCHUNK // N_SUB
SUB = SUB_C * 128


@functools.partial(
    pl.kernel,
    mesh=plsc.VectorSubcoreMesh(core_axis_name="c", subcore_axis_name="s"),
    out_type=jax.ShapeDtypeStruct((2, 2, 16, 16), jnp.float32),
    compiler_params=pltpu.CompilerParams(needs_layout_passes=False),
    scratch_types=[
        pltpu.VMEM((SUB,), jnp.int32),
        pltpu.VMEM((SUB,), jnp.int32),
        pltpu.VMEM((128,), jnp.float32),
        pltpu.VMEM((8000,), jnp.float32),
        pltpu.VMEM((8000,), jnp.float32),
        pltpu.VMEM((16,), jnp.float32),
        pltpu.VMEM_SHARED((TBL,), jnp.float32),
        pltpu.SemaphoreType.DMA,
    ],
)
def _loss_sc(tr, tc, nr, nc, p_hbm, t_hbm, out, rv, cv, vals_v,
             pch_v, tch_v, acc_v, table_sh, sem):
    cid = lax.axis_index("c")
    sid = lax.axis_index("s")

    acc_v[...] = jnp.zeros((16,), jnp.float32)

    @pl.when(sid == 0)
    def _zero_dump():
        pltpu.sync_copy(acc_v, table_sh.at[pl.ds(DUMP, 16)])

    for k in range(8):
        rs = jnp.minimum(sid * 63 + 8 * k, NB - 8)
        off = rs * NB
        pltpu.sync_copy(p_hbm.at[pl.ds(off, 8000)], pch_v)
        pltpu.sync_copy(t_hbm.at[pl.ds(off, 8000)], tch_v)

        def _sq(i, _):
            d = pch_v[pl.ds(i * 16, 16)] - tch_v[pl.ds(i * 16, 16)]
            pch_v[pl.ds(i * 16, 16)] = d * d
            return 0

        lax.fori_loop(0, 500, _sq, 0)
        pltpu.sync_copy(pch_v, table_sh.at[pl.ds(off, 8000)])

    plsc.subcore_barrier()

    wid = sid * 2 + cid

    for a_i, (r_hbm, c_hbm) in enumerate(((tr, tc), (nr, nc))):
        for s_i in range(N_SUB):
            base = wid * TILE_LIN + s_i * SUB
            pltpu.sync_copy(r_hbm.at[pl.ds(base, SUB)], rv)
            pltpu.sync_copy(c_hbm.at[pl.ds(base, SUB)], cv)

            def _linz(g, _):
                rv[pl.ds(g * 16, 16)] = (cv[pl.ds(g * 16, 16)] * NB
                                         + rv[pl.ds(g * 16, 16)])
                return 0

            lax.fori_loop(0, SUB // 16, _linz, 0)

            def _gather(j, _):
                pltpu.async_copy(
                    table_sh.at[rv.at[pl.ds(j * 128, 128)]],
                    vals_v, sem).wait()
                av = acc_v[...]
                for u in range(8):
                    av = av + vals_v[pl.ds(u * 16, 16)]
                acc_v[...] = av
                return 0

            lax.fori_loop(0, SUB_C, _gather, 0)

        pltpu.sync_copy(acc_v, out.at[a_i, cid, sid])
        acc_v[...] = jnp.zeros((16,), jnp.float32)


def _cols(idx):
    r = jnp.pad(idx[:, 0].astype(jnp.int32), (0, LIN_PAD - idx.shape[0]),
                constant_values=0)
    c = jnp.pad(idx[:, 1].astype(jnp.int32), (0, LIN_PAD - idx.shape[0]),
                constant_values=NB)
    return r, c


def kernel(true_index, neg_index, target, preds):
    tr, tc = _cols(true_index)
    nr, nc = _cols(neg_index)
    p = preds.T[:, :NB].reshape(-1)
    t = target.T[:, :NB].reshape(-1)
    parts = _loss_sc(tr, tc, nr, nc, p, t)
    pos = jnp.sum(parts[0])
    neg = jnp.sum(parts[1])
    return (1.0 - 0.5) * pos + 0.5 * neg


# speedup vs baseline: 1.1704x; 1.1704x over previous
import functools

import jax
import jax.numpy as jnp
from jax import lax
from jax.experimental import pallas as pl
from jax.experimental.pallas import tpu as pltpu
from jax.experimental.pallas import tpu_sc as plsc

NB = 1000
DUMP = NB * NB
TBL = NB * NB + 16
N_GCHUNK = 245
TILE_LIN = N_GCHUNK * 128
LIN_PAD = 32 * TILE_LIN


@functools.partial(
    pl.kernel,
    mesh=plsc.VectorSubcoreMesh(core_axis_name="c", subcore_axis_name="s"),
    out_type=jax.ShapeDtypeStruct((2, 2, 16, 16), jnp.float32),
    compiler_params=pltpu.CompilerParams(needs_layout_passes=False),
    scratch_types=[
        pltpu.VMEM((TILE_LIN,), jnp.int32),
        pltpu.VMEM((128,), jnp.float32),
        pltpu.VMEM((8000,), jnp.float32),
        pltpu.VMEM((8000,), jnp.float32),
        pltpu.VMEM((16,), jnp.float32),
        pltpu.VMEM_SHARED((TBL,), jnp.float32),
        pltpu.SemaphoreType.DMA,
    ],
)
def _loss_sc(lint, linn, p_hbm, t_hbm, out, lin_v, vals_v,
             pch_v, tch_v, acc_v, table_sh, sem):
    cid = lax.axis_index("c")
    sid = lax.axis_index("s")

    acc_v[...] = jnp.zeros((16,), jnp.float32)

    @pl.when(sid == 0)
    def _zero_dump():
        pltpu.sync_copy(acc_v, table_sh.at[pl.ds(DUMP, 16)])

    for k in range(8):
        rs = jnp.minimum(sid * 63 + 8 * k, NB - 8)
        off = rs * NB
        pltpu.sync_copy(p_hbm.at[pl.ds(off, 8000)], pch_v)
        pltpu.sync_copy(t_hbm.at[pl.ds(off, 8000)], tch_v)

        def _sq(i, _):
            d = pch_v[pl.ds(i * 16, 16)] - tch_v[pl.ds(i * 16, 16)]
            pch_v[pl.ds(i * 16, 16)] = d * d
            return 0

        lax.fori_loop(0, 500, _sq, 0)
        pltpu.sync_copy(pch_v, table_sh.at[pl.ds(off, 8000)])

    plsc.subcore_barrier()

    wid = sid * 2 + cid
    base = wid * TILE_LIN

    for a_i, arr in enumerate((lint, linn)):
        pltpu.sync_copy(arr.at[pl.ds(base, TILE_LIN)], lin_v)

        def _gather(j, _):
            pltpu.async_copy(
                table_sh.at[lin_v.at[pl.ds(j * 128, 128)]],
                vals_v, sem).wait()
            av = acc_v[...]
            for u in range(8):
                av = av + vals_v[pl.ds(u * 16, 16)]
            acc_v[...] = av
            return 0

        lax.fori_loop(0, N_GCHUNK, _gather, 0)

        pltpu.sync_copy(acc_v, out.at[a_i, cid, sid])
        acc_v[...] = jnp.zeros((16,), jnp.float32)


def _linearize(idx):
    lin = idx[:, 1].astype(jnp.int32) * NB + idx[:, 0].astype(jnp.int32)
    pad = jnp.full((LIN_PAD - lin.shape[0],), DUMP, jnp.int32)
    return jnp.concatenate([lin, pad])


def kernel(true_index, neg_index, target, preds):
    lint = _linearize(true_index)
    linn = _linearize(neg_index)
    p = preds.T[:, :NB].reshape(-1)
    t = target.T[:, :NB].reshape(-1)
    parts = _loss_sc(lint, linn, p, t)
    pos = jnp.sum(parts[0])
    neg = jnp.sum(parts[1])
    return (1.0 - 0.5) * pos + 0.5 * neg


# speedup vs baseline: 1.5893x; 1.3579x over previous
import functools

import jax
import jax.numpy as jnp
from jax import lax
from jax.experimental import pallas as pl
from jax.experimental.pallas import tpu as pltpu
from jax.experimental.pallas import tpu_sc as plsc

NB = 1000
DUMP = NB * NB
TBL = NB * NB + 16
HALF_C = 124
HALF_N = HALF_C * 128
TILE_LIN = 2 * HALF_N
LIN_PAD = 32 * TILE_LIN


@functools.partial(
    pl.kernel,
    mesh=plsc.VectorSubcoreMesh(core_axis_name="c", subcore_axis_name="s"),
    out_type=jax.ShapeDtypeStruct((2, 2, 16, 16), jnp.float32),
    compiler_params=pltpu.CompilerParams(needs_layout_passes=False),
    scratch_types=[
        pltpu.VMEM((TILE_LIN,), jnp.int32),
        pltpu.VMEM((128,), jnp.float32),
        pltpu.VMEM((128,), jnp.float32),
        pltpu.VMEM((8000,), jnp.float32),
        pltpu.VMEM((8000,), jnp.float32),
        pltpu.VMEM((8000,), jnp.float32),
        pltpu.VMEM((8000,), jnp.float32),
        pltpu.VMEM((16,), jnp.float32),
        pltpu.VMEM_SHARED((TBL,), jnp.float32),
        pltpu.SemaphoreType.DMA,
        pltpu.SemaphoreType.DMA,
        pltpu.SemaphoreType.DMA,
        pltpu.SemaphoreType.DMA,
        pltpu.SemaphoreType.DMA,
    ],
)
def _loss_sc(lint, linn, p_hbm, t_hbm, out, lin_v, vals0, vals1,
             pch0, pch1, tch0, tch1, zz_v, table_sh,
             sA0, sA1, sL, sG0, sG1):
    cid = lax.axis_index("c")
    sid = lax.axis_index("s")
    wid = sid * 2 + cid
    base = wid * TILE_LIN

    pch = (pch0, pch1)
    tch = (tch0, tch1)
    sA = (sA0, sA1)

    h_lin = pltpu.async_copy(lint.at[pl.ds(base, HALF_N)],
                             lin_v.at[pl.ds(0, HALF_N)], sL)

    zz_v[...] = jnp.zeros((16,), jnp.float32)

    @pl.when(sid == 0)
    def _zero_dump():
        pltpu.sync_copy(zz_v, table_sh.at[pl.ds(DUMP, 16)])

    def _off(k):
        rs = jnp.minimum(sid * 63 + 8 * k, NB - 8)
        return rs * NB

    hs = [None] * 8
    hs[0] = (pltpu.async_copy(p_hbm.at[pl.ds(_off(0), 8000)], pch[0], sA[0]),
             pltpu.async_copy(t_hbm.at[pl.ds(_off(0), 8000)], tch[0], sA[0]))
    for k in range(8):
        b = k % 2
        if k + 1 < 8:
            nb_ = (k + 1) % 2
            hs[k + 1] = (
                pltpu.async_copy(p_hbm.at[pl.ds(_off(k + 1), 8000)],
                                 pch[nb_], sA[nb_]),
                pltpu.async_copy(t_hbm.at[pl.ds(_off(k + 1), 8000)],
                                 tch[nb_], sA[nb_]))
        hs[k][0].wait()
        hs[k][1].wait()

        @plsc.parallel_loop(0, 500, unroll=4)
        def _sq(i):
            d = pch[b][pl.ds(i * 16, 16)] - tch[b][pl.ds(i * 16, 16)]
            pch[b][pl.ds(i * 16, 16)] = d * d

        pltpu.sync_copy(pch[b], table_sh.at[pl.ds(_off(k), 8000)])

    plsc.subcore_barrier()

    halves = [(0, 0), (0, 1), (1, 0), (1, 1)]

    def _src(arr):
        return lint if arr == 0 else linn

    acc = {0: jnp.zeros((16,), jnp.float32), 1: jnp.zeros((16,), jnp.float32)}
    for hi, (a_i, h) in enumerate(halves):
        h_lin.wait()
        lb = h * HALF_N
        if hi + 1 < 4:
            na, nh = halves[hi + 1]
            h_lin = pltpu.async_copy(
                _src(na).at[pl.ds(wid * TILE_LIN + nh * HALF_N, HALF_N)],
                lin_v.at[pl.ds(nh * HALF_N, HALF_N)], sL)

        def _idx(c):
            return lin_v.at[pl.ds(lb + c * 128, 128)]

        pltpu.async_copy(table_sh.at[_idx(0)], vals0, sG0)
        pltpu.async_copy(table_sh.at[_idx(1)], vals1, sG1)

        def _pair(jj, av):
            pltpu.make_async_copy(table_sh.at[_idx(2 * jj)], vals0,
                                  sG0).wait()
            for u in range(8):
                av = av + vals0[pl.ds(u * 16, 16)]

            @pl.when(jj < HALF_C // 2 - 1)
            def _fire0():
                pltpu.async_copy(table_sh.at[_idx(2 * jj + 2)], vals0, sG0)

            pltpu.make_async_copy(table_sh.at[_idx(2 * jj + 1)], vals1,
                                  sG1).wait()
            for u in range(8):
                av = av + vals1[pl.ds(u * 16, 16)]

            @pl.when(jj < HALF_C // 2 - 1)
            def _fire1():
                pltpu.async_copy(table_sh.at[_idx(2 * jj + 3)], vals1, sG1)

            return av

        acc[a_i] = lax.fori_loop(0, HALF_C // 2, _pair, acc[a_i])

    for a_i in range(2):
        zz_v[...] = acc[a_i]
        pltpu.sync_copy(zz_v, out.at[a_i, cid, sid])


def _linearize(idx):
    lin = idx[:, 1].astype(jnp.int32) * NB + idx[:, 0].astype(jnp.int32)
    pad = jnp.full((LIN_PAD - lin.shape[0],), DUMP, jnp.int32)
    return jnp.concatenate([lin, pad])


def kernel(true_index, neg_index, target, preds):
    lint = _linearize(true_index)
    linn = _linearize(neg_index)
    p = preds.T[:, :NB].reshape(-1)
    t = target.T[:, :NB].reshape(-1)
    parts = _loss_sc(lint, linn, p, t)
    pos = jnp.sum(parts[0])
    neg = jnp.sum(parts[1])
    return (1.0 - 0.5) * pos + 0.5 * neg


# speedup vs baseline: 1.6370x; 1.0300x over previous
import functools

import jax
import jax.numpy as jnp
from jax import lax
from jax.experimental import pallas as pl
from jax.experimental.pallas import tpu as pltpu
from jax.experimental.pallas import tpu_sc as plsc

NB = 1000
DUMP = NB * NB
TBL = 1000448
STG = TBL // 16
STG_C = STG // 8
HALF_C = 124
HALF_N = HALF_C * 128
TILE_LIN = 2 * HALF_N
LIN_PAD = 32 * TILE_LIN

_MESH = plsc.VectorSubcoreMesh(core_axis_name="c", subcore_axis_name="s")


@functools.partial(
    pl.kernel,
    mesh=_MESH,
    out_type=jax.ShapeDtypeStruct((TBL,), jnp.float32),
    compiler_params=pltpu.CompilerParams(needs_layout_passes=False),
    scratch_types=[
        pltpu.VMEM((8000,), jnp.float32),
        pltpu.VMEM((8000,), jnp.float32),
        pltpu.VMEM((8000,), jnp.float32),
        pltpu.VMEM((8000,), jnp.float32),
        pltpu.VMEM((16,), jnp.float32),
        pltpu.SemaphoreType.DMA,
        pltpu.SemaphoreType.DMA,
    ],
)
def _build_sc(p_hbm, t_hbm, out, pch0, pch1, tch0, tch1, zz_v, sA0, sA1):
    cid = lax.axis_index("c")
    sid = lax.axis_index("s")
    pch = (pch0, pch1)
    tch = (tch0, tch1)
    sA = (sA0, sA1)

    zz_v[...] = jnp.zeros((16,), jnp.float32)

    @pl.when(jnp.logical_and(cid == 0, sid == 0))
    def _zero_dump():
        pltpu.sync_copy(zz_v, out.at[pl.ds(DUMP, 16)])

    def _off(k):
        col = jnp.minimum(cid * 500 + sid * 32 + 8 * k, cid * 500 + 492)
        return col * NB

    hs = [None] * 4
    hs[0] = (pltpu.async_copy(p_hbm.at[pl.ds(_off(0), 8000)], pch[0], sA[0]),
             pltpu.async_copy(t_hbm.at[pl.ds(_off(0), 8000)], tch[0], sA[0]))
    for k in range(4):
        b = k % 2
        if k + 1 < 4:
            nb_ = (k + 1) % 2
            hs[k + 1] = (
                pltpu.async_copy(p_hbm.at[pl.ds(_off(k + 1), 8000)],
                                 pch[nb_], sA[nb_]),
                pltpu.async_copy(t_hbm.at[pl.ds(_off(k + 1), 8000)],
                                 tch[nb_], sA[nb_]))
        hs[k][0].wait()
        hs[k][1].wait()

        @plsc.parallel_loop(0, 500, unroll=4)
        def _sq(i):
            d = pch[b][pl.ds(i * 16, 16)] - tch[b][pl.ds(i * 16, 16)]
            pch[b][pl.ds(i * 16, 16)] = d * d

        pltpu.sync_copy(pch[b], out.at[pl.ds(_off(k), 8000)])


@functools.partial(
    pl.kernel,
    mesh=_MESH,
    out_type=jax.ShapeDtypeStruct((2, 2, 16, 16), jnp.float32),
    compiler_params=pltpu.CompilerParams(needs_layout_passes=False),
    scratch_types=[
        pltpu.VMEM((TILE_LIN,), jnp.int32),
        pltpu.VMEM((128,), jnp.float32),
        pltpu.VMEM((128,), jnp.float32),
        pltpu.VMEM((16,), jnp.float32),
        pltpu.VMEM((STG_C,), jnp.float32),
        pltpu.VMEM((STG_C,), jnp.float32),
        pltpu.VMEM_SHARED((TBL,), jnp.float32),
        pltpu.SemaphoreType.DMA,
        pltpu.SemaphoreType.DMA,
        pltpu.SemaphoreType.DMA,
        pltpu.SemaphoreType.DMA,
        pltpu.SemaphoreType.DMA,
    ],
)
def _gather_sc(tbl_hbm, lint, linn, out, lin_v, vals0, vals1, zz_v,
               stg0, stg1, table_sh, sL, sT0, sT1, sG0, sG1):
    cid = lax.axis_index("c")
    sid = lax.axis_index("s")
    wid = sid * 2 + cid
    base = wid * TILE_LIN

    h_lin = pltpu.async_copy(lint.at[pl.ds(base, HALF_N)],
                             lin_v.at[pl.ds(0, HALF_N)], sL)

    stg = (stg0, stg1)
    sT = (sT0, sT1)
    hq = [None] * 8
    hq[0] = pltpu.async_copy(tbl_hbm.at[pl.ds(sid * STG, STG_C)],
                             stg[0], sT[0])
    for q in range(8):
        b = q % 2
        if q + 1 < 8:
            hq[q + 1] = pltpu.async_copy(
                tbl_hbm.at[pl.ds(sid * STG + (q + 1) * STG_C, STG_C)],
                stg[(q + 1) % 2], sT[(q + 1) % 2])
        hq[q].wait()
        pltpu.sync_copy(stg[b], table_sh.at[pl.ds(sid * STG + q * STG_C,
                                                  STG_C)])
    plsc.subcore_barrier()

    halves = [(0, 0), (0, 1), (1, 0), (1, 1)]

    def _src(arr):
        return lint if arr == 0 else linn

    acc = {0: jnp.zeros((16,), jnp.float32), 1: jnp.zeros((16,), jnp.float32)}
    for hi, (a_i, h) in enumerate(halves):
        h_lin.wait()
        lb = h * HALF_N
        if hi + 1 < 4:
            na, nh = halves[hi + 1]
            h_lin = pltpu.async_copy(
                _src(na).at[pl.ds(wid * TILE_LIN + nh * HALF_N, HALF_N)],
                lin_v.at[pl.ds(nh * HALF_N, HALF_N)], sL)

        def _idx(c):
            return lin_v.at[pl.ds(lb + c * 128, 128)]

        pltpu.async_copy(table_sh.at[_idx(0)], vals0, sG0)
        pltpu.async_copy(table_sh.at[_idx(1)], vals1, sG1)

        def _pair(jj, av):
            pltpu.make_async_copy(table_sh.at[_idx(2 * jj)], vals0,
                                  sG0).wait()
            for u in range(8):
                av = av + vals0[pl.ds(u * 16, 16)]

            @pl.when(jj < HALF_C // 2 - 1)
            def _fire0():
                pltpu.async_copy(table_sh.at[_idx(2 * jj + 2)], vals0, sG0)

            pltpu.make_async_copy(table_sh.at[_idx(2 * jj + 1)], vals1,
                                  sG1).wait()
            for u in range(8):
                av = av + vals1[pl.ds(u * 16, 16)]

            @pl.when(jj < HALF_C // 2 - 1)
            def _fire1():
                pltpu.async_copy(table_sh.at[_idx(2 * jj + 3)], vals1, sG1)

            return av

        acc[a_i] = lax.fori_loop(0, HALF_C // 2, _pair, acc[a_i])

    for a_i in range(2):
        zz_v[...] = acc[a_i]
        pltpu.sync_copy(zz_v, out.at[a_i, cid, sid])


def _linearize(idx):
    lin = idx[:, 1].astype(jnp.int32) * NB + idx[:, 0].astype(jnp.int32)
    pad = jnp.full((LIN_PAD - lin.shape[0],), DUMP, jnp.int32)
    return jnp.concatenate([lin, pad])


def kernel(true_index, neg_index, target, preds):
    lint = _linearize(true_index)
    linn = _linearize(neg_index)
    p = preds.T[:, :NB].reshape(-1)
    t = target.T[:, :NB].reshape(-1)
    tbl = _build_sc(p, t)
    parts = _gather_sc(tbl, lint, linn)
    pos = jnp.sum(parts[0])
    neg = jnp.sum(parts[1])
    return (1.0 - 0.5) * pos + 0.5 * neg


# speedup vs baseline: 1.8662x; 1.1400x over previous
import functools

import jax
import jax.numpy as jnp
from jax import lax
from jax.experimental import pallas as pl
from jax.experimental.pallas import tpu as pltpu
from jax.experimental.pallas import tpu_sc as plsc

NB = 1000
DUMP = NB * NB
TBL = 1000448
STG = TBL // 16
STG_C = STG // 8
HALF_C = 124
HALF_N = HALF_C * 128
TILE_LIN = 2 * HALF_N
LIN_PAD = 32 * TILE_LIN

_MESH = plsc.VectorSubcoreMesh(core_axis_name="c", subcore_axis_name="s")


@functools.partial(
    pl.kernel,
    mesh=_MESH,
    out_type=jax.ShapeDtypeStruct((TBL,), jnp.float32),
    compiler_params=pltpu.CompilerParams(needs_layout_passes=False),
    scratch_types=[
        pltpu.VMEM((8000,), jnp.float32),
        pltpu.VMEM((8000,), jnp.float32),
        pltpu.VMEM((8000,), jnp.float32),
        pltpu.VMEM((8000,), jnp.float32),
        pltpu.VMEM((16,), jnp.float32),
        pltpu.SemaphoreType.DMA,
        pltpu.SemaphoreType.DMA,
    ],
)
def _build_sc(p_hbm, t_hbm, out, pch0, pch1, tch0, tch1, zz_v, sA0, sA1):
    cid = lax.axis_index("c")
    sid = lax.axis_index("s")
    pch = (pch0, pch1)
    tch = (tch0, tch1)
    sA = (sA0, sA1)

    zz_v[...] = jnp.zeros((16,), jnp.float32)

    @pl.when(jnp.logical_and(cid == 0, sid == 0))
    def _zero_dump():
        pltpu.sync_copy(zz_v, out.at[pl.ds(DUMP, 16)])

    def _off(k):
        col = jnp.minimum(cid * 500 + sid * 32 + 8 * k, cid * 500 + 492)
        return col * NB

    hs = [None] * 4
    hs[0] = (pltpu.async_copy(p_hbm.at[pl.ds(_off(0), 8000)], pch[0], sA[0]),
             pltpu.async_copy(t_hbm.at[pl.ds(_off(0), 8000)], tch[0], sA[0]))
    for k in range(4):
        b = k % 2
        if k + 1 < 4:
            nb_ = (k + 1) % 2
            hs[k + 1] = (
                pltpu.async_copy(p_hbm.at[pl.ds(_off(k + 1), 8000)],
                                 pch[nb_], sA[nb_]),
                pltpu.async_copy(t_hbm.at[pl.ds(_off(k + 1), 8000)],
                                 tch[nb_], sA[nb_]))
        hs[k][0].wait()
        hs[k][1].wait()

        @plsc.parallel_loop(0, 500, unroll=4)
        def _sq(i):
            d = pch[b][pl.ds(i * 16, 16)] - tch[b][pl.ds(i * 16, 16)]
            pch[b][pl.ds(i * 16, 16)] = d * d

        pltpu.sync_copy(pch[b], out.at[pl.ds(_off(k), 8000)])


@functools.partial(
    pl.kernel,
    mesh=_MESH,
    out_type=jax.ShapeDtypeStruct((2, 16, 16), jnp.float32),
    compiler_params=pltpu.CompilerParams(needs_layout_passes=False),
    scratch_types=[
        pltpu.VMEM((TILE_LIN,), jnp.int32),
        pltpu.VMEM((128,), jnp.float32),
        pltpu.VMEM((128,), jnp.float32),
        pltpu.VMEM((16,), jnp.float32),
        pltpu.VMEM((STG_C,), jnp.float32),
        pltpu.VMEM((STG_C,), jnp.float32),
        pltpu.VMEM_SHARED((TBL,), jnp.float32),
        pltpu.SemaphoreType.DMA,
        pltpu.SemaphoreType.DMA,
        pltpu.SemaphoreType.DMA,
        pltpu.SemaphoreType.DMA,
        pltpu.SemaphoreType.DMA,
    ],
)
def _gather_sc(tbl_hbm, lint, out, lin_v, vals0, vals1, zz_v,
               stg0, stg1, table_sh, sL, sT0, sT1, sG0, sG1):
    cid = lax.axis_index("c")
    sid = lax.axis_index("s")
    wid = sid * 2 + cid
    base = wid * TILE_LIN

    h_lin = pltpu.async_copy(lint.at[pl.ds(base, HALF_N)],
                             lin_v.at[pl.ds(0, HALF_N)], sL)

    stg = (stg0, stg1)
    sT = (sT0, sT1)
    hq = [None] * 8
    hq[0] = pltpu.async_copy(tbl_hbm.at[pl.ds(sid * STG, STG_C)],
                             stg[0], sT[0])
    for q in range(8):
        b = q % 2
        if q + 1 < 8:
            hq[q + 1] = pltpu.async_copy(
                tbl_hbm.at[pl.ds(sid * STG + (q + 1) * STG_C, STG_C)],
                stg[(q + 1) % 2], sT[(q + 1) % 2])
        hq[q].wait()
        pltpu.sync_copy(stg[b], table_sh.at[pl.ds(sid * STG + q * STG_C,
                                                  STG_C)])
    plsc.subcore_barrier()

    acc = jnp.zeros((16,), jnp.float32)
    for h in range(2):
        h_lin.wait()
        lb = h * HALF_N
        if h == 0:
            h_lin = pltpu.async_copy(
                lint.at[pl.ds(wid * TILE_LIN + HALF_N, HALF_N)],
                lin_v.at[pl.ds(HALF_N, HALF_N)], sL)

        def _idx(c):
            return lin_v.at[pl.ds(lb + c * 128, 128)]

        pltpu.async_copy(table_sh.at[_idx(0)], vals0, sG0)
        pltpu.async_copy(table_sh.at[_idx(1)], vals1, sG1)

        def _pair(jj, av):
            pltpu.make_async_copy(table_sh.at[_idx(2 * jj)], vals0,
                                  sG0).wait()
            for u in range(8):
                av = av + vals0[pl.ds(u * 16, 16)]

            @pl.when(jj < HALF_C // 2 - 1)
            def _fire0():
                pltpu.async_copy(table_sh.at[_idx(2 * jj + 2)], vals0, sG0)

            pltpu.make_async_copy(table_sh.at[_idx(2 * jj + 1)], vals1,
                                  sG1).wait()
            for u in range(8):
                av = av + vals1[pl.ds(u * 16, 16)]

            @pl.when(jj < HALF_C // 2 - 1)
            def _fire1():
                pltpu.async_copy(table_sh.at[_idx(2 * jj + 3)], vals1, sG1)

            return av

        acc = lax.fori_loop(0, HALF_C // 2, _pair, acc)

    zz_v[...] = acc
    pltpu.sync_copy(zz_v, out.at[cid, sid])


def _linearize(idx):
    lin = idx[:, 1].astype(jnp.int32) * NB + idx[:, 0].astype(jnp.int32)
    pad = jnp.full((LIN_PAD - lin.shape[0],), DUMP, jnp.int32)
    return jnp.concatenate([lin, pad])


def kernel(true_index, neg_index, target, preds):
    lint = _linearize(true_index)
    linn = _linearize(neg_index)
    p = preds.T[:, :NB].reshape(-1)
    t = target.T[:, :NB].reshape(-1)
    tbl = _build_sc(p, t)
    pos = jnp.sum(_gather_sc(tbl, lint))
    neg = jnp.sum(_gather_sc(tbl, linn))
    return (1.0 - 0.5) * pos + 0.5 * neg


# speedup vs baseline: 1.9674x; 1.0542x over previous
import functools

import jax
import jax.numpy as jnp
from jax import lax
from jax.experimental import pallas as pl
from jax.experimental.pallas import tpu as pltpu
from jax.experimental.pallas import tpu_sc as plsc

NB = 1000
DUMP = NB * NB
TBL = 1000448
STG = TBL // 16
STG_C = STG // 8
HALF_C = 124
HALF_N = HALF_C * 128
QSRC = 500000
LIN_PAD = 32 * HALF_N

_MESH = plsc.VectorSubcoreMesh(core_axis_name="c", subcore_axis_name="s")


@functools.partial(
    pl.kernel,
    mesh=_MESH,
    out_type=jax.ShapeDtypeStruct((TBL,), jnp.float32),
    compiler_params=pltpu.CompilerParams(needs_layout_passes=False),
    scratch_types=[
        pltpu.VMEM((8000,), jnp.float32),
        pltpu.VMEM((8000,), jnp.float32),
        pltpu.VMEM((8000,), jnp.float32),
        pltpu.VMEM((8000,), jnp.float32),
        pltpu.VMEM((16,), jnp.float32),
        pltpu.SemaphoreType.DMA,
        pltpu.SemaphoreType.DMA,
    ],
)
def _build_sc(p_hbm, t_hbm, out, pch0, pch1, tch0, tch1, zz_v, sA0, sA1):
    cid = lax.axis_index("c")
    sid = lax.axis_index("s")
    pch = (pch0, pch1)
    tch = (tch0, tch1)
    sA = (sA0, sA1)

    zz_v[...] = jnp.zeros((16,), jnp.float32)

    @pl.when(jnp.logical_and(cid == 0, sid == 0))
    def _zero_dump():
        pltpu.sync_copy(zz_v, out.at[pl.ds(DUMP, 16)])

    def _off(k):
        col = jnp.minimum(cid * 500 + sid * 32 + 8 * k, cid * 500 + 492)
        return col * NB

    hs = [None] * 4
    hs[0] = (pltpu.async_copy(p_hbm.at[pl.ds(_off(0), 8000)], pch[0], sA[0]),
             pltpu.async_copy(t_hbm.at[pl.ds(_off(0), 8000)], tch[0], sA[0]))
    for k in range(4):
        b = k % 2
        if k + 1 < 4:
            nb_ = (k + 1) % 2
            hs[k + 1] = (
                pltpu.async_copy(p_hbm.at[pl.ds(_off(k + 1), 8000)],
                                 pch[nb_], sA[nb_]),
                pltpu.async_copy(t_hbm.at[pl.ds(_off(k + 1), 8000)],
                                 tch[nb_], sA[nb_]))
        hs[k][0].wait()
        hs[k][1].wait()

        @plsc.parallel_loop(0, 500, unroll=4)
        def _sq(i):
            d = pch[b][pl.ds(i * 16, 16)] - tch[b][pl.ds(i * 16, 16)]
            pch[b][pl.ds(i * 16, 16)] = d * d

        pltpu.sync_copy(pch[b], out.at[pl.ds(_off(k), 8000)])


@functools.partial(
    pl.kernel,
    mesh=_MESH,
    out_type=jax.ShapeDtypeStruct((2, 16, 16), jnp.float32),
    compiler_params=pltpu.CompilerParams(needs_layout_passes=False),
    scratch_types=[
        pltpu.VMEM((HALF_N,), jnp.int32),
        pltpu.VMEM((128,), jnp.float32),
        pltpu.VMEM((128,), jnp.float32),
        pltpu.VMEM((16,), jnp.float32),
        pltpu.VMEM((STG_C,), jnp.float32),
        pltpu.VMEM((STG_C,), jnp.float32),
        pltpu.VMEM_SHARED((TBL,), jnp.float32),
        pltpu.SemaphoreType.DMA,
        pltpu.SemaphoreType.DMA,
        pltpu.SemaphoreType.DMA,
        pltpu.SemaphoreType.DMA,
        pltpu.SemaphoreType.DMA,
    ],
)
def _gather_sc(tbl_hbm, lint, out, lin_v, vals0, vals1, zz_v,
               stg0, stg1, table_sh, sL, sT0, sT1, sG0, sG1):
    cid = lax.axis_index("c")
    sid = lax.axis_index("s")
    wid = sid * 2 + cid
    base = wid * HALF_N

    h_lin = pltpu.async_copy(lint.at[pl.ds(base, HALF_N)], lin_v, sL)

    stg = (stg0, stg1)
    sT = (sT0, sT1)
    hq = [None] * 8
    hq[0] = pltpu.async_copy(tbl_hbm.at[pl.ds(sid * STG, STG_C)],
                             stg[0], sT[0])
    for q in range(8):
        b = q % 2
        if q + 1 < 8:
            hq[q + 1] = pltpu.async_copy(
                tbl_hbm.at[pl.ds(sid * STG + (q + 1) * STG_C, STG_C)],
                stg[(q + 1) % 2], sT[(q + 1) % 2])
        hq[q].wait()
        pltpu.sync_copy(stg[b], table_sh.at[pl.ds(sid * STG + q * STG_C,
                                                  STG_C)])
    plsc.subcore_barrier()

    acc = jnp.zeros((16,), jnp.float32)
    for h in range(1):
        h_lin.wait()

        def _idx(c):
            return lin_v.at[pl.ds(c * 128, 128)]

        pltpu.async_copy(table_sh.at[_idx(0)], vals0, sG0)
        pltpu.async_copy(table_sh.at[_idx(1)], vals1, sG1)

        def _pair(jj, av):
            pltpu.make_async_copy(table_sh.at[_idx(2 * jj)], vals0,
                                  sG0).wait()
            for u in range(8):
                av = av + vals0[pl.ds(u * 16, 16)]

            @pl.when(jj < HALF_C // 2 - 1)
            def _fire0():
                pltpu.async_copy(table_sh.at[_idx(2 * jj + 2)], vals0, sG0)

            pltpu.make_async_copy(table_sh.at[_idx(2 * jj + 1)], vals1,
                                  sG1).wait()
            for u in range(8):
                av = av + vals1[pl.ds(u * 16, 16)]

            @pl.when(jj < HALF_C // 2 - 1)
            def _fire1():
                pltpu.async_copy(table_sh.at[_idx(2 * jj + 3)], vals1, sG1)

            return av

        acc = lax.fori_loop(0, HALF_C // 2, _pair, acc)

    zz_v[...] = acc
    pltpu.sync_copy(zz_v, out.at[cid, sid])


def _linearize(idx, lo, hi):
    sl = idx[lo:hi]
    lin = sl[:, 1].astype(jnp.int32) * NB + sl[:, 0].astype(jnp.int32)
    pad = jnp.full((LIN_PAD - lin.shape[0],), DUMP, jnp.int32)
    return jnp.concatenate([lin, pad])


def kernel(true_index, neg_index, target, preds):
    quarters = [
        _linearize(true_index, 0, QSRC),
        _linearize(true_index, QSRC, 2 * QSRC),
        _linearize(neg_index, 0, QSRC),
        _linearize(neg_index, QSRC, 2 * QSRC),
    ]
    p = preds.T[:, :NB].reshape(-1)
    t = target.T[:, :NB].reshape(-1)
    tbl = _build_sc(p, t)
    sums = [jnp.sum(_gather_sc(tbl, q)) for q in quarters]
    pos = sums[0] + sums[1]
    neg = sums[2] + sums[3]
    return (1.0 - 0.5) * pos + 0.5 * neg
